# baseline (speedup 1.0000x reference)
import functools

import numpy as np

import jax
import jax.numpy as jnp
from jax import lax
from jax.experimental import pallas as pl
from jax.experimental.pallas import tpu as pltpu
from jax.experimental.pallas import tpu_sc as plsc

N = 10000
E = 320000
F = 128
FH = F // 2

NUM_CORES = 2
NUM_SUBCORES = 16
CHUNK = 128
NBUF = 2
GC = 8
CHUNKS_PER_TEC = GC * (-(-E // (NUM_SUBCORES * CHUNK * GC)))
NGROUPS = CHUNKS_PER_TEC // GC
EDGES_PER_TEC = CHUNKS_PER_TEC * CHUNK
E_PAD = EDGES_PER_TEC * NUM_SUBCORES
ROWS_PER_SUBCORE = 640
N_PAD = NUM_SUBCORES * ROWS_PER_SUBCORE


def _dense_body(x_ref, wb_ref, wd_ref, bias_ref, hid_ref, base_ref):
    xb = x_ref[...]
    hid = jnp.dot(xb, wb_ref[...], preferred_element_type=jnp.float32)
    hid_ref[0] = hid[:, :FH].astype(jnp.bfloat16)
    hid_ref[1] = hid[:, FH:].astype(jnp.bfloat16)
    base_ref[...] = (
        jnp.dot(jnp.abs(xb), wd_ref[...], preferred_element_type=jnp.float32)
        + bias_ref[...]
    )


def _dense(x, wb, wd, bias):
    blk = 2000
    return pl.pallas_call(
        _dense_body,
        grid=(N // blk,),
        in_specs=[
            pl.BlockSpec((blk, F), lambda i: (i, 0)),
            pl.BlockSpec((F, F), lambda i: (0, 0)),
            pl.BlockSpec((F, F), lambda i: (0, 0)),
            pl.BlockSpec((1, F), lambda i: (0, 0)),
        ],
        out_specs=[
            pl.BlockSpec((2, blk, FH), lambda i: (0, i, 0)),
            pl.BlockSpec((blk, F), lambda i: (i, 0)),
        ],
        out_shape=[
            jax.ShapeDtypeStruct((2, N_PAD, FH), jnp.bfloat16),
            jax.ShapeDtypeStruct((N, F), jnp.float32),
        ],
    )(x, wb, wd, bias)


def _spmm_body(hid_hbm, idx_hbm, ew_hbm, out_hbm,
               idxr, ewr, colv, rowv, gbuf, sbuf, tbl, accum,
               gsem, ssem, isem, wsem):
    c = lax.axis_index("c")
    s = lax.axis_index("s")
    cpt = CHUNKS_PER_TEC
    base_r = s * ROWS_PER_SUBCORE

    def _zrow(r, _):
        for j in range(FH // 16):
            sbuf[r, pl.ds(j * 16, 16)] = jnp.zeros((16,), jnp.float32)
        return 0
    lax.fori_loop(0, NBUF * CHUNK, _zrow, 0)
    pltpu.sync_copy(sbuf, accum.at[pl.ds(base_r, NBUF * CHUNK)])
    pltpu.sync_copy(sbuf, accum.at[pl.ds(base_r + NBUF * CHUNK, NBUF * CHUNK)])
    pltpu.sync_copy(sbuf.at[pl.ds(0, CHUNK)],
                    accum.at[pl.ds(base_r + 2 * NBUF * CHUNK, CHUNK)])
    pltpu.sync_copy(hid_hbm.at[c, pl.ds(base_r, ROWS_PER_SUBCORE)],
                    tbl.at[pl.ds(base_r, ROWS_PER_SUBCORE)])

    def _stage_group(gi, slot):
        pltpu.async_copy(idx_hbm.at[pl.ds(s * cpt + gi * GC, GC)],
                         idxr.at[slot], isem.at[slot])
        pltpu.async_copy(ew_hbm.at[pl.ds(s * cpt + gi * GC, GC)],
                         ewr.at[slot], wsem.at[slot])

    def _wait_group(slot):
        pltpu.make_async_copy(idx_hbm.at[pl.ds(0, GC)], idxr.at[slot],
                              isem.at[slot]).wait()
        pltpu.make_async_copy(ew_hbm.at[pl.ds(0, GC)], ewr.at[slot],
                              wsem.at[slot]).wait()

    _stage_group(0, 0)
    plsc.subcore_barrier()
    _wait_group(0)

    def _unpack_col(tc, b):
        gslot = (tc // GC) % 2
        off = tc % GC
        for j in range(CHUNK // 16):
            p = idxr[gslot, off, pl.ds(j * 16, 16)]
            colv[b, pl.ds(j * 16, 16)] = lax.bitwise_and(p, 0xFFFF)

    def _unpack_row(tc, b):
        gslot = (tc // GC) % 2
        off = tc % GC
        for j in range(CHUNK // 16):
            p = idxr[gslot, off, pl.ds(j * 16, 16)]
            rowv[b, pl.ds(j * 16, 16)] = lax.shift_right_logical(p, 16)

    def _gather(tc, b):
        _unpack_col(tc, b)
        pltpu.async_copy(tbl.at[colv.at[b]], gbuf.at[pl.ds(b * CHUNK, CHUNK)],
                         gsem.at[b])

    for b in range(NBUF):
        _gather(b, b)

    def _outer(t, _):
        for b in range(NBUF):
            tc = t * NBUF + b

            @pl.when(jnp.logical_and(tc % GC == 0, tc // GC + 1 < NGROUPS))
            def _():
                _stage_group(tc // GC + 1, (tc // GC + 1) % 2)

            @pl.when(jnp.logical_and((tc + NBUF) % GC == 0,
                                     (tc + NBUF) // GC < NGROUPS))
            def _():
                _wait_group(((tc + NBUF) // GC) % 2)

            pltpu.make_async_copy(tbl.at[colv.at[b]],
                                  gbuf.at[pl.ds(b * CHUNK, CHUNK)],
                                  gsem.at[b]).wait()

            @pl.when(t > 0)
            def _():
                pltpu.make_async_copy(sbuf.at[pl.ds(b * CHUNK, CHUNK)],
                                      accum.at[rowv.at[b]], ssem.at[b]).wait()

            def _scale(g, _):
                gslot = (tc // GC) % 2
                off = tc % GC
                wv = ewr[gslot, off, pl.ds(g * 16, 16)]
                for lane in range(16):
                    w = wv[lane]
                    k = b * CHUNK + g * 16 + lane
                    for j in range(FH // 32):
                        ab = gbuf[k, pl.ds(j * 32, 32)]
                        ev, od = plsc.unpack(
                            ab, format=plsc.PackFormat.INTERLEAVED,
                            preferred_element_type=jnp.float32)
                        sbuf[k, pl.ds(j * 32, 16)] = ev * w
                        sbuf[k, pl.ds(j * 32 + 16, 16)] = od * w
                return 0
            lax.fori_loop(0, CHUNK // 16, _scale, 0)

            _unpack_row(tc, b)

            @pl.when(tc + NBUF < cpt)
            def _():
                _gather(tc + NBUF, b)

            pltpu.async_copy(sbuf.at[pl.ds(b * CHUNK, CHUNK)],
                             accum.at[rowv.at[b]], ssem.at[b], add=True)
        return 0
    lax.fori_loop(0, cpt // NBUF, _outer, 0)
    for b in range(NBUF):
        pltpu.make_async_copy(sbuf.at[pl.ds(b * CHUNK, CHUNK)],
                              accum.at[rowv.at[b]], ssem.at[b]).wait()
    plsc.subcore_barrier()

    pltpu.sync_copy(
        accum.at[pl.ds(base_r, ROWS_PER_SUBCORE)],
        out_hbm.at[c, pl.ds(base_r, ROWS_PER_SUBCORE)],
    )


_spmm = functools.partial(
    pl.kernel,
    out_type=jax.ShapeDtypeStruct((NUM_CORES, N_PAD, FH), jnp.float32),
    mesh=plsc.VectorSubcoreMesh(core_axis_name="c", subcore_axis_name="s"),
    compiler_params=pltpu.CompilerParams(use_tc_tiling_on_sc=False,
                                         needs_layout_passes=False),
    scratch_types=[
        pltpu.VMEM((2, GC, CHUNK), jnp.int32),
        pltpu.VMEM((2, GC, CHUNK), jnp.float32),
        pltpu.VMEM((NBUF, CHUNK), jnp.int32),
        pltpu.VMEM((NBUF, CHUNK), jnp.int32),
        pltpu.VMEM((NBUF * CHUNK, FH), jnp.bfloat16),
        pltpu.VMEM((NBUF * CHUNK, FH), jnp.float32),
        pltpu.VMEM_SHARED((N_PAD, FH), jnp.bfloat16),
        pltpu.VMEM_SHARED((N_PAD, FH), jnp.float32),
        pltpu.SemaphoreType.DMA((NBUF,)),
        pltpu.SemaphoreType.DMA((NBUF,)),
        pltpu.SemaphoreType.DMA((2,)),
        pltpu.SemaphoreType.DMA((2,)),
    ],
)(_spmm_body)


def _combine_body(p0_ref, p1_ref, base_ref, out_ref):
    out_ref[...] = (
        jnp.concatenate([p0_ref[0], p1_ref[0]], axis=1) + base_ref[...]
    )


def _combine(partials, base):
    blk = 1000
    return pl.pallas_call(
        _combine_body,
        grid=(N // blk,),
        in_specs=[
            pl.BlockSpec((1, blk, FH), lambda i: (0, i, 0)),
            pl.BlockSpec((1, blk, FH), lambda i: (1, i, 0)),
            pl.BlockSpec((blk, F), lambda i: (i, 0)),
        ],
        out_specs=pl.BlockSpec((blk, F), lambda i: (i, 0)),
        out_shape=jax.ShapeDtypeStruct((N, F), jnp.float32),
    )(partials, partials, base)


_PERM = np.empty(F, dtype=np.int32)
for _h in range(F // 32):
    for _i in range(16):
        _PERM[32 * _h + 2 * _i] = 32 * _h + _i
        _PERM[32 * _h + 2 * _i + 1] = 32 * _h + 16 + _i


def kernel(x, edge_index, edge_weight, w_b, w_a, w_c, b_b, b_a, b_c):
    wd = (w_c - w_a) * (1.0 / 3.0)
    bias = b_b + (b_c - b_a) * (1.0 / 3.0)
    row = edge_index[0].astype(jnp.int32)
    col = edge_index[1].astype(jnp.int32)
    ew = edge_weight.astype(jnp.float32)
    pad = E_PAD - E
    packed = jnp.pad(col | (row << 16), (0, pad)).reshape(-1, CHUNK)
    ew = jnp.pad(ew, (0, pad)).reshape(-1, CHUNK)
    hidden, base = _dense(x, w_b[:, _PERM], wd, bias)
    partials = _spmm(hidden, packed, ew)
    return _combine(partials, base)
